# baseline (speedup 1.0000x reference)
import functools
import math

import jax
import jax.numpy as jnp
from jax.experimental import pallas as pl

_H = 16


def _matmul_bias_kernel(x_ref, w_ref, b_ref, o_ref):
    o_ref[...] = jax.lax.dot_general(
        x_ref[...], w_ref[...], (((1,), (1,)), ((), ())),
        preferred_element_type=jnp.float32,
    ) + b_ref[...]


def _linear(xf, w, b, blk_m):
    m, k = xf.shape
    n = w.shape[0]
    return pl.pallas_call(
        _matmul_bias_kernel,
        grid=(m // blk_m,),
        in_specs=[
            pl.BlockSpec((blk_m, k), lambda i: (i, 0)),
            pl.BlockSpec((n, k), lambda i: (0, 0)),
            pl.BlockSpec((1, n), lambda i: (0, 0)),
        ],
        out_specs=pl.BlockSpec((blk_m, n), lambda i: (i, 0)),
        out_shape=jax.ShapeDtypeStruct((m, n), jnp.float32),
    )(xf, w, b.reshape(1, n))


def _attn_kernel(scale, q_ref, k_ref, v_ref, o_ref):
    qb = q_ref[0, 0]
    kb = k_ref[0, 0]
    vb = v_ref[0, 0]
    logits = jax.lax.dot_general(
        qb, kb, (((1,), (1,)), ((), ())),
        preferred_element_type=jnp.float32,
    ) * scale
    m = jnp.max(logits, axis=1, keepdims=True)
    p = jnp.exp(logits - m)
    s = jnp.sum(p, axis=1, keepdims=True)
    ctx = jax.lax.dot_general(
        p, vb, (((1,), (0,)), ((), ())),
        preferred_element_type=jnp.float32,
    )
    o_ref[0, 0] = ctx / s


def kernel(x, Wqkv, bqkv, Wout, bout):
    b, s, c = x.shape
    h = _H
    d = c // h
    m = b * s

    qkv = _linear(x.reshape(m, c), Wqkv, bqkv, blk_m=512)

    qkv = qkv.reshape(b, s, 3, h, d)
    q = jnp.transpose(qkv[:, :, 0], (0, 2, 1, 3))
    k = jnp.transpose(qkv[:, :, 1], (0, 2, 1, 3))
    v = jnp.transpose(qkv[:, :, 2], (0, 2, 1, 3))

    blk_q = 512
    scale = 1.0 / math.sqrt(d)
    ctx = pl.pallas_call(
        functools.partial(_attn_kernel, scale),
        grid=(b, h, s // blk_q),
        in_specs=[
            pl.BlockSpec((1, 1, blk_q, d), lambda bi, hi, qi: (bi, hi, qi, 0)),
            pl.BlockSpec((1, 1, s, d), lambda bi, hi, qi: (bi, hi, 0, 0)),
            pl.BlockSpec((1, 1, s, d), lambda bi, hi, qi: (bi, hi, 0, 0)),
        ],
        out_specs=pl.BlockSpec((1, 1, blk_q, d), lambda bi, hi, qi: (bi, hi, qi, 0)),
        out_shape=jax.ShapeDtypeStruct((b, h, s, d), jnp.float32),
    )(q, k, v)

    ctxf = jnp.transpose(ctx, (0, 2, 1, 3)).reshape(m, c)
    out = _linear(ctxf, Wout, bout, blk_m=512)
    return out.reshape(b, s, c)
